# baseline (speedup 1.0000x reference)
import functools

import jax
import jax.numpy as jnp
from jax import lax
from jax.experimental import pallas as pl
from jax.experimental.pallas import tpu as pltpu
from jax.experimental.pallas import tpu_sc as plsc

_NC = 2
_NS = 16
_NW = _NC * _NS

_CHUNK = 512
_NBUF = 3


def _make_gather(n_rows, vocab, d):
    n_per_w = n_rows // _NW
    n_chunks = n_per_w // _CHUNK
    mesh = plsc.VectorSubcoreMesh(core_axis_name="c", subcore_axis_name="s")

    @functools.partial(
        pl.kernel,
        mesh=mesh,
        compiler_params=pltpu.CompilerParams(use_tc_tiling_on_sc=False),
        out_type=jax.ShapeDtypeStruct((n_rows, d), jnp.float32),
        scratch_types=[
            pltpu.VMEM((n_per_w,), jnp.int32),
            pltpu.VMEM((_NBUF, _CHUNK, d), jnp.float32),
            pltpu.SemaphoreType.DMA((_NBUF,)),
            pltpu.SemaphoreType.DMA((_NBUF,)),
        ],
    )
    def k(ids_hbm, table_hbm, out_flat, idx_v, rows_v, gsem, wsem):
        cid = lax.axis_index("c")
        sid = lax.axis_index("s")
        wid = sid * _NC + cid
        base = wid * n_per_w
        pltpu.sync_copy(ids_hbm.at[pl.ds(base, n_per_w)], idx_v)

        def issue_gather(j, b):
            return pltpu.async_copy(
                table_hbm.at[idx_v.at[pl.ds(j * _CHUNK, _CHUNK)]],
                rows_v.at[b], gsem.at[b]
            )

        def issue_write(j, b):
            return pltpu.async_copy(
                rows_v.at[b],
                out_flat.at[pl.ds(base + j * _CHUNK, _CHUNK)],
                wsem.at[b],
            )

        gh = {}
        wh = {}
        gh[0] = issue_gather(0, 0)
        gh[1] = issue_gather(1, 1)
        for j in range(n_chunks):
            b = j % _NBUF
            gh.pop(j).wait()
            wh[j] = issue_write(j, b)
            jn = j + 2
            if jn < n_chunks:
                bn = jn % _NBUF
                if jn >= _NBUF:
                    wh.pop(jn - _NBUF).wait()
                gh[jn] = issue_gather(jn, bn)
        for j in sorted(wh):
            wh.pop(j).wait()

    return k


_TC = 512


def _make_transpose(vocab, d):
    n_chunks = vocab // _TC + (1 if vocab % _TC else 0)
    per_w = n_chunks // _NW + (1 if n_chunks % _NW else 0)
    mesh = plsc.VectorSubcoreMesh(core_axis_name="c", subcore_axis_name="s")

    @functools.partial(
        pl.kernel,
        mesh=mesh,
        compiler_params=pltpu.CompilerParams(
            use_tc_tiling_on_sc=False, needs_layout_passes=False
        ),
        out_type=jax.ShapeDtypeStruct((vocab, d), jnp.float32),
        scratch_types=[
            pltpu.VMEM((d * _TC,), jnp.float32),
            pltpu.VMEM((_TC, d), jnp.float32),
            pltpu.SemaphoreType.DMA,
        ],
    )
    def k(tt_hbm, out_hbm, in_f, out_v, lsem):
        cid = lax.axis_index("c")
        sid = lax.axis_index("s")
        wid = sid * _NC + cid
        kvecs = [
            (lax.iota(jnp.int32, 16) + 16 * kk) * _TC for kk in range(d // 16)
        ]

        def chunk_body(i, carry):
            c = wid + i * _NW

            @pl.when(c < n_chunks)
            def _():
                col0 = jnp.minimum(c * _TC, vocab - _TC)
                hs = [
                    pltpu.async_copy(
                        tt_hbm.at[cc, pl.ds(col0, _TC)],
                        in_f.at[pl.ds(cc * _TC, _TC)],
                        lsem,
                    )
                    for cc in range(d)
                ]
                for h in hs:
                    h.wait()

                def row_body(ro, rcarry):
                    for ri in range(4):
                        r = ro * 4 + ri
                        rvec = jnp.full((16,), r, jnp.int32)
                        for kk in range(d // 16):
                            vals = plsc.load_gather(in_f, [kvecs[kk] + rvec])
                            out_v[r, pl.ds(16 * kk, 16)] = vals
                    return rcarry

                lax.fori_loop(0, _TC // 4, row_body, 0)
                pltpu.sync_copy(out_v, out_hbm.at[pl.ds(col0, _TC)])

            return carry

        lax.fori_loop(0, per_w, chunk_body, 0)

    return k


def kernel(input_ids, embeddings):
    b, s = input_ids.shape
    vocab, d = embeddings.shape
    n = b * s
    ids = input_ids.reshape(n).astype(jnp.int32)
    table = _make_transpose(vocab, d)(embeddings.T)
    out = _make_gather(n, vocab, d)(ids, table)
    return out.reshape(b, s, d)


# speedup vs baseline: 5.1448x; 5.1448x over previous
import functools

import jax
import jax.numpy as jnp
from jax import lax
from jax.experimental import pallas as pl
from jax.experimental.pallas import tpu as pltpu
from jax.experimental.pallas import tpu_sc as plsc

_NC = 2
_NS = 16
_NW = _NC * _NS
_LB = 128
_PAD = 65


def _make_gather(batch, seq, vocab, d):
    n_per_w = (batch // _NW) * seq
    mesh = plsc.VectorSubcoreMesh(core_axis_name="c", subcore_axis_name="s")

    @functools.partial(
        pl.kernel,
        mesh=mesh,
        compiler_params=pltpu.CompilerParams(
            use_tc_tiling_on_sc=False, needs_layout_passes=False
        ),
        out_type=jax.ShapeDtypeStruct((seq, d // 8, _NW, 8, _LB), jnp.float32),
        scratch_types=[
            pltpu.VMEM((n_per_w,), jnp.int32),
            pltpu.VMEM((2, _LB), jnp.int32),
            pltpu.VMEM((2, _LB, d), jnp.float32),
            pltpu.VMEM((_LB * _PAD,), jnp.float32),
            pltpu.VMEM((2, d // 8, 8, _LB), jnp.float32),
            pltpu.SemaphoreType.DMA((2,)),
            pltpu.SemaphoreType.DMA((2,)),
        ],
    )
    def k(ids_hbm, table_hbm, out5, idx_v, bidx, gbuf, pbuf, wbuf, gsem, wsem):
        cid = lax.axis_index("c")
        sid = lax.axis_index("s")
        wid = sid * _NC + cid
        pltpu.sync_copy(ids_hbm.at[pl.ds(wid * n_per_w, n_per_w)], idx_v)

        iota = lax.iota(jnp.int32, 16)
        pvecs = [(iota + 16 * g) * seq for g in range(_LB // 16)]
        lvecs = [(iota + 16 * g) * _PAD for g in range(_LB // 16)]

        def build_bidx(s_val, bsel):
            sv = jnp.full((16,), s_val, jnp.int32)
            for g in range(_LB // 16):
                bidx[bsel, pl.ds(16 * g, 16)] = plsc.load_gather(
                    idx_v, [pvecs[g] + sv]
                )

        def issue_gather(bsel):
            return pltpu.async_copy(
                table_hbm.at[bidx.at[bsel]], gbuf.at[bsel], gsem.at[bsel]
            )

        def wait_gather(bsel):
            pltpu.make_async_copy(
                table_hbm.at[bidx.at[bsel]], gbuf.at[bsel], gsem.at[bsel]
            ).wait()

        def issue_write(s_val, bsel):
            return pltpu.async_copy(
                wbuf.at[bsel], out5.at[s_val, :, wid], wsem.at[bsel]
            )

        def wait_write(bsel):
            pltpu.make_async_copy(
                wbuf.at[bsel], out5.at[0, :, wid], wsem.at[bsel]
            ).wait()

        def phase(s_val, bsel):
            nxt = s_val + 1

            @pl.when(nxt < seq)
            def _():
                build_bidx(nxt, 1 - bsel)

                @pl.when(nxt >= 2)
                def _():
                    wait_write(1 - bsel)

                issue_gather(1 - bsel)

            wait_gather(bsel)

            def stage_body(r0, carry):
                for ri in range(4):
                    lb = r0 * 4 + ri
                    for kk in range(d // 16):
                        pbuf[pl.ds(lb * _PAD + 16 * kk, 16)] = gbuf[
                            bsel, lb, pl.ds(16 * kk, 16)
                        ]
                return carry

            lax.fori_loop(0, _LB // 4, stage_body, 0)

            def tr_body(d0, carry):
                for di in range(4):
                    dd = d0 * 4 + di
                    dv = jnp.full((16,), dd, jnp.int32)
                    p = dd // 8
                    r = dd % 8
                    for g in range(_LB // 16):
                        wbuf[bsel, p, r, pl.ds(16 * g, 16)] = plsc.load_gather(
                            pbuf, [lvecs[g] + dv]
                        )
                return carry

            lax.fori_loop(0, d // 4, tr_body, 0)
            issue_write(s_val, bsel)

        build_bidx(0, 0)
        issue_gather(0)

        def body(t, carry):
            phase(2 * t, 0)
            phase(2 * t + 1, 1)
            return carry

        lax.fori_loop(0, seq // 2, body, 0)
        wait_write(0)
        wait_write(1)

    return k


def kernel(input_ids, embeddings):
    b, s = input_ids.shape
    vocab, d = embeddings.shape
    n = b * s
    ids = input_ids.reshape(n).astype(jnp.int32)
    out5 = _make_gather(b, s, vocab, d)(ids, embeddings)
    out = out5.transpose(2, 4, 0, 1, 3).reshape(b, s, d)
    return out


# speedup vs baseline: 5.9372x; 1.1540x over previous
import functools

import jax
import jax.numpy as jnp
from jax import lax
from jax.experimental import pallas as pl
from jax.experimental.pallas import tpu as pltpu
from jax.experimental.pallas import tpu_sc as plsc

_NC = 2
_NS = 16
_NW = _NC * _NS

_CHUNK = 512
_NBUF = 3


def _make_gather(n_rows, vocab, d):
    n_per_w = n_rows // _NW
    n_chunks = n_per_w // _CHUNK
    mesh = plsc.VectorSubcoreMesh(core_axis_name="c", subcore_axis_name="s")

    @functools.partial(
        pl.kernel,
        mesh=mesh,
        compiler_params=pltpu.CompilerParams(use_tc_tiling_on_sc=False),
        out_type=jax.ShapeDtypeStruct((n_rows, d), jnp.float32),
        scratch_types=[
            pltpu.VMEM((n_per_w,), jnp.int32),
            pltpu.VMEM((_NBUF, _CHUNK, d), jnp.float32),
            pltpu.SemaphoreType.DMA((_NBUF,)),
            pltpu.SemaphoreType.DMA((_NBUF,)),
        ],
    )
    def k(ids_hbm, table_hbm, out_flat, idx_v, rows_v, gsem, wsem):
        cid = lax.axis_index("c")
        sid = lax.axis_index("s")
        wid = sid * _NC + cid
        base = wid * n_per_w
        pltpu.sync_copy(ids_hbm.at[pl.ds(base, n_per_w)], idx_v)

        def issue_gather(j, b):
            return pltpu.async_copy(
                table_hbm.at[idx_v.at[pl.ds(j * _CHUNK, _CHUNK)]],
                rows_v.at[b], gsem.at[b]
            )

        def issue_write(j, b):
            return pltpu.async_copy(
                rows_v.at[b],
                out_flat.at[pl.ds(base + j * _CHUNK, _CHUNK)],
                wsem.at[b],
            )

        gh = {}
        wh = {}
        gh[0] = issue_gather(0, 0)
        gh[1] = issue_gather(1, 1)
        for j in range(n_chunks):
            b = j % _NBUF
            gh.pop(j).wait()
            wh[j] = issue_write(j, b)
            jn = j + 2
            if jn < n_chunks:
                bn = jn % _NBUF
                if jn >= _NBUF:
                    wh.pop(jn - _NBUF).wait()
                gh[jn] = issue_gather(jn, bn)
        for j in sorted(wh):
            wh.pop(j).wait()

    return k


def kernel(input_ids, embeddings):
    b, s = input_ids.shape
    vocab, d = embeddings.shape
    n = b * s
    ids = input_ids.reshape(n).astype(jnp.int32)
    out = _make_gather(n, vocab, d)(ids, embeddings)
    return out.reshape(b, s, d)
